# Initial kernel scaffold; baseline (speedup 1.0000x reference)
import jax
import jax.numpy as jnp
from jax.experimental import pallas as pl


def kernel(x, edge_index, batch, candidate_edges, params):
    raise NotImplementedError("write your pallas kernel here")


# speedup vs baseline: 3.5023x; 3.5023x over previous
import functools

import jax
import jax.numpy as jnp
from jax import lax
from jax.experimental import pallas as pl
from jax.experimental.pallas import tpu as pltpu
from jax.experimental.pallas import tpu_sc as plsc

N = 10000
E = 320000
D = 128
H = 128
L = 3
G = 64
NCLS = 10
EC = 100000

NSC = 2
NTS = 16
NW = NSC * NTS

EPT = E // NW
KC = 80
NCHUNK = EPT // KC
NACC = 10240
RPT_ACC = NACC // NTS

ECP = 102400
RPT_E = ECP // NW
KE = 128
NECH = RPT_E // KE
PE = 256


def _sc_mesh():
    return plsc.VectorSubcoreMesh(core_axis_name="c", subcore_axis_name="s")


def _agg_body(h_hbm, src_hbm, dst_hbm, out_hbm,
              acc_sh, idxs_v, idxd_v, rows_v, cbuf_v, sem):
    c = lax.axis_index("c")
    s = lax.axis_index("s")
    wid = c * NTS + s

    def _zrow(i, _):
        def _zcol(j, _):
            cbuf_v[i, pl.ds(j * 16, 16)] = jnp.zeros((16,), jnp.float32)
            return 0
        return lax.fori_loop(0, D // 16, _zcol, 0)
    lax.fori_loop(0, 128, _zrow, 0)

    def _zcp(j, _):
        pltpu.sync_copy(cbuf_v, acc_sh.at[pl.ds(s * RPT_ACC + j * 128, 128)])
        return 0
    lax.fori_loop(0, RPT_ACC // 128, _zcp, 0)
    plsc.subcore_barrier()

    base = wid * EPT

    def _chunk(j, _):
        off = base + j * KC
        pltpu.sync_copy(src_hbm.at[pl.ds(off, KC)], idxs_v)
        pltpu.sync_copy(dst_hbm.at[pl.ds(off, KC)], idxd_v)
        pltpu.async_copy(h_hbm.at[idxs_v], rows_v, sem).wait()
        pltpu.sync_copy(rows_v, acc_sh.at[idxd_v], add=True)
        return 0
    lax.fori_loop(0, NCHUNK, _chunk, 0)
    plsc.subcore_barrier()

    def _ocp(j, _):
        r0 = s * RPT_ACC + j * 128
        pltpu.sync_copy(acc_sh.at[pl.ds(r0, 128)], cbuf_v)
        pltpu.sync_copy(cbuf_v, out_hbm.at[c].at[pl.ds(r0, 128)])
        return 0
    lax.fori_loop(0, RPT_ACC // 128, _ocp, 0)


def _sc_aggregate(h, src, dst):
    fn = pl.kernel(
        _agg_body,
        out_type=jax.ShapeDtypeStruct((NSC, NACC, D), jnp.float32),
        mesh=_sc_mesh(),
        scratch_types=[
            pltpu.VMEM_SHARED((NACC, D), jnp.float32),
            pltpu.VMEM((KC,), jnp.int32),
            pltpu.VMEM((KC,), jnp.int32),
            pltpu.VMEM((KC, D), jnp.float32),
            pltpu.VMEM((128, D), jnp.float32),
            pltpu.SemaphoreType.DMA,
        ],
    )
    return fn(h, src, dst)


def _edge_body(p_hbm, q_hbm, sidx_hbm, didx_hbm, out_hbm,
               idxs_v, idxd_v, bufp_v, bufq_v, semp, semq):
    c = lax.axis_index("c")
    s = lax.axis_index("s")
    wid = c * NTS + s
    base = wid * RPT_E

    def _chunk(j, _):
        off = base + j * KE
        pltpu.sync_copy(sidx_hbm.at[pl.ds(off, KE)], idxs_v)
        pltpu.sync_copy(didx_hbm.at[pl.ds(off, KE)], idxd_v)
        cp1 = pltpu.async_copy(p_hbm.at[idxs_v], bufp_v, semp)
        cp2 = pltpu.async_copy(q_hbm.at[idxd_v], bufq_v, semq)
        cp1.wait()
        cp2.wait()

        def _row(i, _):
            for k in range(PE // 16):
                sl = pl.ds(k * 16, 16)
                bufp_v[i, sl] = jnp.maximum(bufp_v[i, sl] + bufq_v[i, sl], 0.0)
            return 0
        lax.fori_loop(0, KE, _row, 0)
        pltpu.sync_copy(bufp_v, out_hbm.at[pl.ds(off, KE)])
        return 0
    lax.fori_loop(0, NECH, _chunk, 0)


def _sc_edge_gather(p, q, sidx, didx):
    fn = pl.kernel(
        _edge_body,
        out_type=jax.ShapeDtypeStruct((ECP, PE), jnp.float32),
        mesh=_sc_mesh(),
        scratch_types=[
            pltpu.VMEM((KE,), jnp.int32),
            pltpu.VMEM((KE,), jnp.int32),
            pltpu.VMEM((KE, PE), jnp.float32),
            pltpu.VMEM((KE, PE), jnp.float32),
            pltpu.SemaphoreType.DMA,
            pltpu.SemaphoreType.DMA,
        ],
    )
    return fn(p, q, sidx, didx)


BM = 1000
NBLK = N // BM


def _gin_mlp_body(eps_ref, h_ref, a0_ref, a1_ref, w1_ref, b1_ref,
                  w2_ref, b2_ref, z_ref, st_ref):
    z0 = h_ref[...] * (1.0 + eps_ref[0, 0]) + a0_ref[...] + a1_ref[...]
    z1 = jnp.maximum(
        jnp.dot(z0, w1_ref[...], preferred_element_type=jnp.float32)
        + b1_ref[...], 0.0)
    z2 = jnp.maximum(
        jnp.dot(z1, w2_ref[...], preferred_element_type=jnp.float32)
        + b2_ref[...], 0.0)
    z_ref[...] = z2

    @pl.when(pl.program_id(0) == 0)
    def _():
        st_ref[...] = jnp.zeros_like(st_ref)

    st_ref[0:1, :] += jnp.sum(z2, axis=0, keepdims=True)
    st_ref[1:2, :] += jnp.sum(z2 * z2, axis=0, keepdims=True)


def _tc_gin_mlp(eps, h, a0, a1, w1, b1, w2, b2):
    full = lambda shape: pl.BlockSpec(shape, lambda i: (0, 0))
    return pl.pallas_call(
        _gin_mlp_body,
        grid=(NBLK,),
        in_specs=[
            full((1, 1)),
            pl.BlockSpec((BM, D), lambda i: (i, 0)),
            pl.BlockSpec((BM, D), lambda i: (i, 0)),
            pl.BlockSpec((BM, D), lambda i: (i, 0)),
            full((D, H)), full((1, H)), full((H, H)), full((1, H)),
        ],
        out_specs=[
            pl.BlockSpec((BM, H), lambda i: (i, 0)),
            full((8, H)),
        ],
        out_shape=[
            jax.ShapeDtypeStruct((N, H), jnp.float32),
            jax.ShapeDtypeStruct((8, H), jnp.float32),
        ],
    )(eps, h, a0, a1, w1, b1, w2, b2)


def _norm_body(z_ref, st_ref, g_ref, b_ref, h_ref):
    inv_n = 1.0 / N
    mu = st_ref[0:1, :] * inv_n
    var = st_ref[1:2, :] * inv_n - mu * mu
    inv = lax.rsqrt(var + 1e-5)
    h_ref[...] = (z_ref[...] - mu) * (g_ref[...] * inv) + b_ref[...]


def _tc_norm(z, st, gamma, beta):
    full = lambda shape: pl.BlockSpec(shape, lambda i: (0, 0))
    return pl.pallas_call(
        _norm_body,
        grid=(NBLK,),
        in_specs=[
            pl.BlockSpec((BM, H), lambda i: (i, 0)),
            full((8, H)), full((1, H)), full((1, H)),
        ],
        out_specs=pl.BlockSpec((BM, H), lambda i: (i, 0)),
        out_shape=jax.ShapeDtypeStruct((N, H), jnp.float32),
    )(z, st, gamma, beta)


def _pool_cls_body(emb_ref, b_ref, w0, b0, w1, b1, w2, b2, w3, b3,
                   out_ref, sums_ref, cnt_ref):
    i = pl.program_id(0)

    @pl.when(i == 0)
    def _():
        sums_ref[...] = jnp.zeros_like(sums_ref)
        cnt_ref[...] = jnp.zeros_like(cnt_ref)

    seg = b_ref[...]
    iota = lax.broadcasted_iota(jnp.int32, (BM, G), 1)
    onehot = (seg == iota).astype(jnp.float32)
    sums_ref[...] += lax.dot_general(
        onehot, emb_ref[...], (((0,), (0,)), ((), ())),
        preferred_element_type=jnp.float32)
    cnt_ref[...] += jnp.sum(onehot, axis=0, keepdims=True)

    @pl.when(i == NBLK - 1)
    def _():
        cnt = jnp.maximum(cnt_ref[...], 1.0)
        gemb = sums_ref[...] / jnp.transpose(cnt)
        g = jnp.maximum(
            jnp.dot(gemb, w0[...], preferred_element_type=jnp.float32)
            + b0[...], 0.0)
        g = jnp.maximum(
            jnp.dot(g, w1[...], preferred_element_type=jnp.float32)
            + b1[...], 0.0)
        g = jnp.maximum(
            jnp.dot(g, w2[...], preferred_element_type=jnp.float32)
            + b2[...], 0.0)
        g = jnp.dot(g, w3[...], preferred_element_type=jnp.float32) + b3[...]
        m = jnp.max(g, axis=1, keepdims=True)
        eg = jnp.exp(g - m)
        out_ref[...] = (g - m) - jnp.log(jnp.sum(eg, axis=1, keepdims=True))


def _tc_pool_cls(emb, batch_col, ncls):
    (w0, b0), (w1, b1), (w2, b2), (w3, b3) = ncls
    full = lambda shape: pl.BlockSpec(shape, lambda i: (0, 0))
    return pl.pallas_call(
        _pool_cls_body,
        grid=(NBLK,),
        in_specs=[
            pl.BlockSpec((BM, L * H), lambda i: (i, 0)),
            pl.BlockSpec((BM, 1), lambda i: (i, 0)),
            full((L * H, 2 * H)), full((1, 2 * H)),
            full((2 * H, H)), full((1, H)),
            full((H, H)), full((1, H)),
            full((H, NCLS)), full((1, NCLS)),
        ],
        out_specs=full((G, NCLS)),
        out_shape=jax.ShapeDtypeStruct((G, NCLS), jnp.float32),
        scratch_shapes=[
            pltpu.VMEM((G, L * H), jnp.float32),
            pltpu.VMEM((1, G), jnp.float32),
        ],
    )(emb, batch_col, w0, b0.reshape(1, -1), w1, b1.reshape(1, -1),
      w2, b2.reshape(1, -1), w3, b3.reshape(1, -1))


def _pq_body(emb_ref, wa_ref, wb_ref, b1_ref, p_ref, q_ref):
    e = emb_ref[...]
    p_ref[...] = jnp.dot(e, wa_ref[...], preferred_element_type=jnp.float32)
    q_ref[...] = jnp.dot(e, wb_ref[...],
                         preferred_element_type=jnp.float32) + b1_ref[...]


def _tc_pq(emb, wa, wb, b1):
    full = lambda shape: pl.BlockSpec(shape, lambda i: (0, 0))
    return pl.pallas_call(
        _pq_body,
        grid=(NBLK,),
        in_specs=[
            pl.BlockSpec((BM, L * H), lambda i: (i, 0)),
            full((L * H, PE)), full((L * H, PE)), full((1, PE)),
        ],
        out_specs=[
            pl.BlockSpec((BM, PE), lambda i: (i, 0)),
            pl.BlockSpec((BM, PE), lambda i: (i, 0)),
        ],
        out_shape=[
            jax.ShapeDtypeStruct((N, PE), jnp.float32),
            jax.ShapeDtypeStruct((N, PE), jnp.float32),
        ],
    )(emb, wa, wb, b1)


EBM = 1000
NEBLK = EC // EBM


def _edge_mlp_body(r_ref, w2_ref, b2_ref, w3_ref, b3_ref, out_ref):
    h2 = jnp.maximum(
        jnp.dot(r_ref[...], w2_ref[...], preferred_element_type=jnp.float32)
        + b2_ref[...], 0.0)
    sc = jnp.dot(h2, w3_ref[...],
                 preferred_element_type=jnp.float32) + b3_ref[0, 0]
    out_ref[...] = 1.0 / (1.0 + jnp.exp(-sc))


def _tc_edge_mlp(r, w2, b2, w3, b3):
    full = lambda shape: pl.BlockSpec(shape, lambda i: (0, 0))
    return pl.pallas_call(
        _edge_mlp_body,
        grid=(NEBLK,),
        in_specs=[
            pl.BlockSpec((EBM, PE), lambda i: (i, 0)),
            full((PE, H)), full((1, H)), full((H, 1)), full((1, 1)),
        ],
        out_specs=pl.BlockSpec((EBM, 1), lambda i: (i, 0)),
        out_shape=jax.ShapeDtypeStruct((EC, 1), jnp.float32),
    )(r, w2, b2.reshape(1, -1), w3, b3.reshape(1, 1))


def kernel(x, edge_index, batch, candidate_edges, params):
    src = edge_index[0]
    dst = edge_index[1]

    h = x
    xs = []
    for p in params["convs"]:
        agg = _sc_aggregate(h, src, dst)
        eps = p["eps"].reshape(1, 1)
        z, st = _tc_gin_mlp(eps, h, agg[0], agg[1],
                            p["w1"], p["b1"].reshape(1, -1),
                            p["w2"], p["b2"].reshape(1, -1))
        h = _tc_norm(z, st, p["gamma"].reshape(1, -1),
                     p["beta"].reshape(1, -1))
        xs.append(h)
    node_emb = jnp.concatenate(xs, axis=1)

    class_out = _tc_pool_cls(node_emb, batch.reshape(N, 1),
                             params["node_cls"])

    (w1e, b1e), (w2e, b2e), (w3e, b3e) = params["edge_pred"]
    p_arr, q_arr = _tc_pq(node_emb, w1e[: L * H], w1e[L * H:], b1e.reshape(1, -1))

    pad = jnp.arange(ECP - EC, dtype=jnp.int32) % N
    sidx = jnp.concatenate([candidate_edges[0], pad])
    didx = jnp.concatenate([candidate_edges[1], pad])
    r = _sc_edge_gather(p_arr, q_arr, sidx, didx)

    scores = _tc_edge_mlp(r, w2e, b2e, w3e, b3e)
    return (class_out, scores[:, 0])


# speedup vs baseline: 6.5997x; 1.8844x over previous
import functools

import jax
import jax.numpy as jnp
from jax import lax
from jax.experimental import pallas as pl
from jax.experimental.pallas import tpu as pltpu
from jax.experimental.pallas import tpu_sc as plsc

N = 10000
E = 320000
D = 128
H = 128
L = 3
G = 64
NCLS = 10
EC = 100000

NSC = 2
NTS = 16
NW = NSC * NTS

EPT = E // NW
KC = 80
NCHUNK = EPT // KC
NBUF_A = 2
NACC = 10240
RPT_ACC = NACC // NTS

ECP = 102400
RPT_E = ECP // NW
KE = 64
NECH = RPT_E // KE
NBUF_E = 2
PE = 256


def _sc_mesh():
    return plsc.VectorSubcoreMesh(core_axis_name="c", subcore_axis_name="s")


def _agg_body(h_hbm, src_hbm, dst_hbm, out_hbm,
              acc_sh, idxs_v, d0_v, d1_v, r0_v, r1_v,
              g0, g1, s0, s1, i0, i1):
    c = lax.axis_index("c")
    s = lax.axis_index("s")
    wid = c * NTS + s
    rows = (r0_v, r1_v)
    dbuf = (d0_v, d1_v)
    gsem = (g0, g1)
    ssem = (s0, s1)
    isem = (i0, i1)

    pltpu.sync_copy(src_hbm.at[wid], idxs_v)

    def _zrow(i, _):
        for j in range(D // 16):
            r0_v[i, pl.ds(j * 16, 16)] = jnp.zeros((16,), jnp.float32)
        return 0
    lax.fori_loop(0, KC, _zrow, 0)
    base_r = s * RPT_ACC
    for k in range(RPT_ACC // KC):
        pltpu.sync_copy(r0_v, acc_sh.at[pl.ds(base_r + k * KC, KC)])
    plsc.subcore_barrier()

    for b in range(NBUF_A):
        pltpu.async_copy(dst_hbm.at[wid, b], dbuf[b], isem[b])
        pltpu.async_copy(h_hbm.at[idxs_v.at[b]], rows[b], gsem[b])

    def _round(j, _):
        for b in range(NBUF_A):
            jj = NBUF_A * j + b
            pltpu.make_async_copy(
                dst_hbm.at[wid, 0], dbuf[b], isem[b]).wait()
            pltpu.make_async_copy(
                h_hbm.at[idxs_v.at[0]], rows[b], gsem[b]).wait()
            pltpu.async_copy(rows[b], acc_sh.at[dbuf[b]], ssem[b], add=True)

            @pl.when(jj + NBUF_A < NCHUNK)
            def _():
                pltpu.make_async_copy(
                    rows[b], acc_sh.at[dbuf[b]], ssem[b]).wait()
                pltpu.async_copy(dst_hbm.at[wid, jj + NBUF_A], dbuf[b],
                                 isem[b])
                pltpu.async_copy(
                    h_hbm.at[idxs_v.at[jj + NBUF_A]], rows[b], gsem[b])
        return 0
    lax.fori_loop(0, NCHUNK // NBUF_A, _round, 0)
    pltpu.make_async_copy(dst_hbm.at[wid, 0], dbuf[0], isem[0]).wait()
    pltpu.make_async_copy(h_hbm.at[idxs_v.at[0]], rows[0], gsem[0]).wait()
    pltpu.async_copy(rows[0], acc_sh.at[dbuf[0]], ssem[0], add=True)
    for b in range(NBUF_A):
        pltpu.make_async_copy(rows[b], acc_sh.at[dbuf[b]], ssem[b]).wait()
    plsc.subcore_barrier()

    pltpu.sync_copy(acc_sh.at[pl.ds(base_r, RPT_ACC)],
                    out_hbm.at[c].at[pl.ds(base_r, RPT_ACC)])


def _sc_aggregate(h, src, dst):
    fn = pl.kernel(
        _agg_body,
        out_type=jax.ShapeDtypeStruct((NSC, NACC, D), jnp.float32),
        mesh=_sc_mesh(),
        scratch_types=[
            pltpu.VMEM_SHARED((NACC, D), jnp.float32),
            pltpu.VMEM((NCHUNK, KC), jnp.int32),
            pltpu.VMEM((KC,), jnp.int32),
            pltpu.VMEM((KC,), jnp.int32),
            pltpu.VMEM((KC, D), jnp.float32),
            pltpu.VMEM((KC, D), jnp.float32),
        ] + [pltpu.SemaphoreType.DMA] * 6,
    )
    return fn(h, src.reshape(NW, NCHUNK, KC), dst.reshape(NW, NCHUNK, KC))


def _edge_body(p_hbm, q_hbm, sidx_hbm, didx_hbm, out_hbm,
               idxs_v, idxd_v, a0_v, a1_v, b0_v, b1_v,
               gp0, gp1, gq0, gq1, w0, w1):
    c = lax.axis_index("c")
    s = lax.axis_index("s")
    wid = c * NTS + s
    base = wid * RPT_E
    abuf = (a0_v, a1_v)
    bbuf = (b0_v, b1_v)
    gpsem = (gp0, gp1)
    gqsem = (gq0, gq1)
    wsem = (w0, w1)

    pltpu.sync_copy(sidx_hbm.at[wid], idxs_v)
    pltpu.sync_copy(didx_hbm.at[wid], idxd_v)

    for b in range(NBUF_E):
        pltpu.async_copy(p_hbm.at[idxs_v.at[b]], abuf[b], gpsem[b])
        pltpu.async_copy(q_hbm.at[idxd_v.at[b]], bbuf[b], gqsem[b])

    def _round(j, _):
        for b in range(NBUF_E):
            jj = NBUF_E * j + b
            pltpu.make_async_copy(
                p_hbm.at[idxs_v.at[0]], abuf[b], gpsem[b]).wait()
            pltpu.make_async_copy(
                q_hbm.at[idxd_v.at[0]], bbuf[b], gqsem[b]).wait()

            def _row(i, _):
                for k in range(PE // 16):
                    sl = pl.ds(k * 16, 16)
                    abuf[b][i, sl] = jnp.maximum(
                        abuf[b][i, sl] + bbuf[b][i, sl], 0.0)
                return 0
            lax.fori_loop(0, KE, _row, 0)
            pltpu.async_copy(
                abuf[b], out_hbm.at[pl.ds(base + jj * KE, KE)], wsem[b])

            @pl.when(jj + NBUF_E < NECH)
            def _():
                pltpu.make_async_copy(
                    abuf[b], out_hbm.at[pl.ds(base, KE)], wsem[b]).wait()
                pltpu.async_copy(
                    p_hbm.at[idxs_v.at[jj + NBUF_E]], abuf[b], gpsem[b])
                pltpu.async_copy(
                    q_hbm.at[idxd_v.at[jj + NBUF_E]], bbuf[b], gqsem[b])
        return 0
    lax.fori_loop(0, NECH // NBUF_E, _round, 0)
    for b in range(NBUF_E):
        pltpu.make_async_copy(
            abuf[b], out_hbm.at[pl.ds(base, KE)], wsem[b]).wait()


def _sc_edge_gather(p, q, sidx, didx):
    fn = pl.kernel(
        _edge_body,
        out_type=jax.ShapeDtypeStruct((ECP, PE), jnp.float32),
        mesh=_sc_mesh(),
        scratch_types=[
            pltpu.VMEM((NECH, KE), jnp.int32),
            pltpu.VMEM((NECH, KE), jnp.int32),
            pltpu.VMEM((KE, PE), jnp.float32),
            pltpu.VMEM((KE, PE), jnp.float32),
            pltpu.VMEM((KE, PE), jnp.float32),
            pltpu.VMEM((KE, PE), jnp.float32),
        ] + [pltpu.SemaphoreType.DMA] * 6,
    )
    return fn(p, q, sidx.reshape(NW, NECH, KE), didx.reshape(NW, NECH, KE))


BM = 1000
NBLK = N // BM


def _gin_mlp_body(eps_ref, h_ref, a0_ref, a1_ref, w1_ref, b1_ref,
                  w2_ref, b2_ref, z_ref, st_ref):
    z0 = h_ref[...] * (1.0 + eps_ref[0, 0]) + a0_ref[...] + a1_ref[...]
    z1 = jnp.maximum(
        jnp.dot(z0, w1_ref[...], preferred_element_type=jnp.float32)
        + b1_ref[...], 0.0)
    z2 = jnp.maximum(
        jnp.dot(z1, w2_ref[...], preferred_element_type=jnp.float32)
        + b2_ref[...], 0.0)
    z_ref[...] = z2

    @pl.when(pl.program_id(0) == 0)
    def _():
        st_ref[...] = jnp.zeros_like(st_ref)

    st_ref[0:1, :] += jnp.sum(z2, axis=0, keepdims=True)
    st_ref[1:2, :] += jnp.sum(z2 * z2, axis=0, keepdims=True)


def _tc_gin_mlp(eps, h, a0, a1, w1, b1, w2, b2):
    full = lambda shape: pl.BlockSpec(shape, lambda i: (0, 0))
    return pl.pallas_call(
        _gin_mlp_body,
        grid=(NBLK,),
        in_specs=[
            full((1, 1)),
            pl.BlockSpec((BM, D), lambda i: (i, 0)),
            pl.BlockSpec((BM, D), lambda i: (i, 0)),
            pl.BlockSpec((BM, D), lambda i: (i, 0)),
            full((D, H)), full((1, H)), full((H, H)), full((1, H)),
        ],
        out_specs=[
            pl.BlockSpec((BM, H), lambda i: (i, 0)),
            full((8, H)),
        ],
        out_shape=[
            jax.ShapeDtypeStruct((N, H), jnp.float32),
            jax.ShapeDtypeStruct((8, H), jnp.float32),
        ],
    )(eps, h, a0, a1, w1, b1, w2, b2)


def _norm_body(z_ref, st_ref, g_ref, b_ref, h_ref):
    inv_n = 1.0 / N
    mu = st_ref[0:1, :] * inv_n
    var = st_ref[1:2, :] * inv_n - mu * mu
    inv = lax.rsqrt(var + 1e-5)
    h_ref[...] = (z_ref[...] - mu) * (g_ref[...] * inv) + b_ref[...]


def _tc_norm(z, st, gamma, beta):
    full = lambda shape: pl.BlockSpec(shape, lambda i: (0, 0))
    return pl.pallas_call(
        _norm_body,
        grid=(NBLK,),
        in_specs=[
            pl.BlockSpec((BM, H), lambda i: (i, 0)),
            full((8, H)), full((1, H)), full((1, H)),
        ],
        out_specs=pl.BlockSpec((BM, H), lambda i: (i, 0)),
        out_shape=jax.ShapeDtypeStruct((N, H), jnp.float32),
    )(z, st, gamma, beta)


def _pool_cls_body(emb_ref, b_ref, w0, b0, w1, b1, w2, b2, w3, b3,
                   out_ref, sums_ref, cnt_ref):
    i = pl.program_id(0)

    @pl.when(i == 0)
    def _():
        sums_ref[...] = jnp.zeros_like(sums_ref)
        cnt_ref[...] = jnp.zeros_like(cnt_ref)

    seg = b_ref[...]
    iota = lax.broadcasted_iota(jnp.int32, (BM, G), 1)
    onehot = (seg == iota).astype(jnp.float32)
    sums_ref[...] += lax.dot_general(
        onehot, emb_ref[...], (((0,), (0,)), ((), ())),
        preferred_element_type=jnp.float32)
    cnt_ref[...] += jnp.sum(onehot, axis=0, keepdims=True)

    @pl.when(i == NBLK - 1)
    def _():
        cnt = jnp.maximum(cnt_ref[...], 1.0)
        gemb = sums_ref[...] / jnp.transpose(cnt)
        g = jnp.maximum(
            jnp.dot(gemb, w0[...], preferred_element_type=jnp.float32)
            + b0[...], 0.0)
        g = jnp.maximum(
            jnp.dot(g, w1[...], preferred_element_type=jnp.float32)
            + b1[...], 0.0)
        g = jnp.maximum(
            jnp.dot(g, w2[...], preferred_element_type=jnp.float32)
            + b2[...], 0.0)
        g = jnp.dot(g, w3[...], preferred_element_type=jnp.float32) + b3[...]
        m = jnp.max(g, axis=1, keepdims=True)
        eg = jnp.exp(g - m)
        out_ref[...] = (g - m) - jnp.log(jnp.sum(eg, axis=1, keepdims=True))


def _tc_pool_cls(emb, batch_col, ncls):
    (w0, b0), (w1, b1), (w2, b2), (w3, b3) = ncls
    full = lambda shape: pl.BlockSpec(shape, lambda i: (0, 0))
    return pl.pallas_call(
        _pool_cls_body,
        grid=(NBLK,),
        in_specs=[
            pl.BlockSpec((BM, L * H), lambda i: (i, 0)),
            pl.BlockSpec((BM, 1), lambda i: (i, 0)),
            full((L * H, 2 * H)), full((1, 2 * H)),
            full((2 * H, H)), full((1, H)),
            full((H, H)), full((1, H)),
            full((H, NCLS)), full((1, NCLS)),
        ],
        out_specs=full((G, NCLS)),
        out_shape=jax.ShapeDtypeStruct((G, NCLS), jnp.float32),
        scratch_shapes=[
            pltpu.VMEM((G, L * H), jnp.float32),
            pltpu.VMEM((1, G), jnp.float32),
        ],
    )(emb, batch_col, w0, b0.reshape(1, -1), w1, b1.reshape(1, -1),
      w2, b2.reshape(1, -1), w3, b3.reshape(1, -1))


def _pq_body(emb_ref, wa_ref, wb_ref, b1_ref, p_ref, q_ref):
    e = emb_ref[...]
    p_ref[...] = jnp.dot(e, wa_ref[...], preferred_element_type=jnp.float32)
    q_ref[...] = jnp.dot(e, wb_ref[...],
                         preferred_element_type=jnp.float32) + b1_ref[...]


def _tc_pq(emb, wa, wb, b1):
    full = lambda shape: pl.BlockSpec(shape, lambda i: (0, 0))
    return pl.pallas_call(
        _pq_body,
        grid=(NBLK,),
        in_specs=[
            pl.BlockSpec((BM, L * H), lambda i: (i, 0)),
            full((L * H, PE)), full((L * H, PE)), full((1, PE)),
        ],
        out_specs=[
            pl.BlockSpec((BM, PE), lambda i: (i, 0)),
            pl.BlockSpec((BM, PE), lambda i: (i, 0)),
        ],
        out_shape=[
            jax.ShapeDtypeStruct((N, PE), jnp.float32),
            jax.ShapeDtypeStruct((N, PE), jnp.float32),
        ],
    )(emb, wa, wb, b1)


EBM = 1000
NEBLK = EC // EBM


def _edge_mlp_body(r_ref, w2_ref, b2_ref, w3_ref, b3_ref, out_ref):
    h2 = jnp.maximum(
        jnp.dot(r_ref[...], w2_ref[...], preferred_element_type=jnp.float32)
        + b2_ref[...], 0.0)
    sc = jnp.dot(h2, w3_ref[...],
                 preferred_element_type=jnp.float32) + b3_ref[0, 0]
    out_ref[...] = 1.0 / (1.0 + jnp.exp(-sc))


def _tc_edge_mlp(r, w2, b2, w3, b3):
    full = lambda shape: pl.BlockSpec(shape, lambda i: (0, 0))
    return pl.pallas_call(
        _edge_mlp_body,
        grid=(NEBLK,),
        in_specs=[
            pl.BlockSpec((EBM, PE), lambda i: (i, 0)),
            full((PE, H)), full((1, H)), full((H, 1)), full((1, 1)),
        ],
        out_specs=pl.BlockSpec((EBM, 1), lambda i: (i, 0)),
        out_shape=jax.ShapeDtypeStruct((EC, 1), jnp.float32),
    )(r, w2, b2.reshape(1, -1), w3, b3.reshape(1, 1))


def kernel(x, edge_index, batch, candidate_edges, params):
    src = edge_index[0]
    dst = edge_index[1]

    h = x
    xs = []
    for p in params["convs"]:
        agg = _sc_aggregate(h, src, dst)
        eps = p["eps"].reshape(1, 1)
        z, st = _tc_gin_mlp(eps, h, agg[0], agg[1],
                            p["w1"], p["b1"].reshape(1, -1),
                            p["w2"], p["b2"].reshape(1, -1))
        h = _tc_norm(z, st, p["gamma"].reshape(1, -1),
                     p["beta"].reshape(1, -1))
        xs.append(h)
    node_emb = jnp.concatenate(xs, axis=1)

    class_out = _tc_pool_cls(node_emb, batch.reshape(N, 1),
                             params["node_cls"])

    (w1e, b1e), (w2e, b2e), (w3e, b3e) = params["edge_pred"]
    p_arr, q_arr = _tc_pq(node_emb, w1e[: L * H], w1e[L * H:], b1e.reshape(1, -1))

    pad = jnp.arange(ECP - EC, dtype=jnp.int32) % N
    sidx = jnp.concatenate([candidate_edges[0], pad])
    didx = jnp.concatenate([candidate_edges[1], pad])
    r = _sc_edge_gather(p_arr, q_arr, sidx, didx)

    scores = _tc_edge_mlp(r, w2e, b2e, w3e, b3e)
    return (class_out, scores[:, 0])
